# baseline (speedup 1.0000x reference)
import functools

import numpy as np
import jax
import jax.numpy as jnp
from jax.experimental import pallas as pl
from jax.experimental.pallas import tpu as pltpu


def _phases(f):
    out = []
    for k in range(f):
        d = (k + 0.5) / f - 0.5
        if d < 0:
            out.append((True, -d, 1.0 + d))
        else:
            out.append((False, 1.0 - d, d))
    return out


def _up_rows(x, f):
    if f == 1:
        return x
    xprev = jnp.concatenate([x[:1], x[:-1]], axis=0)
    xnext = jnp.concatenate([x[1:], x[-1:]], axis=0)
    cols = []
    for use_prev, wa, wb in _phases(f):
        cols.append(wa * xprev + wb * x if use_prev else wa * x + wb * xnext)
    y = jnp.stack(cols, axis=1)
    return y.reshape(x.shape[0] * f, x.shape[1], x.shape[2])


def _interp_matrix(out_size, in_size):
    if out_size == in_size:
        return np.eye(out_size, dtype=np.float32)
    scale = in_size / out_size
    i = np.arange(out_size, dtype=np.float64)
    src = np.maximum((i + 0.5) * scale - 0.5, 0.0)
    i0 = np.minimum(np.floor(src).astype(np.int64), in_size - 1)
    i1 = np.minimum(i0 + 1, in_size - 1)
    w1 = src - i0
    r = np.zeros((out_size, in_size), dtype=np.float64)
    r[np.arange(out_size), i0] += 1.0 - w1
    r[np.arange(out_size), i1] += w1
    return r.astype(np.float32)


def _add_up_cols(slab, y, bd_ref, G, W, H, ws, C):
    yb = y.astype(jnp.bfloat16).reshape(H * ws, C)
    for g in range(H // G):
        zg = jnp.dot(bd_ref[...], yb[g * G * ws:(g + 1) * G * ws, :],
                     preferred_element_type=jnp.float32)
        rows = slice(g * G, (g + 1) * G)
        slab[rows, :, :] = slab[rows, :, :] + zg.reshape(G, W, C)


def _conv3x3_tile(spad, w_ref, r0, th, W, C):
    acc = jnp.zeros((th * W, C), jnp.float32)
    for dh in range(3):
        for dw in range(3):
            k = dh * 3 + dw
            xs = spad[pl.ds(r0 + dh, th), dw:dw + W, :].reshape(th * W, C)
            acc = acc + jnp.dot(xs, w_ref[k * C:(k + 1) * C, :],
                                preferred_element_type=jnp.float32)
    return acc


def _decoder_kernel(x0_ref, x1_ref, x2_ref, p0_ref, p1_ref, p2_ref,
                    bsum_ref, w1_ref, b1_ref, w2_ref, b2_ref, bd1_ref,
                    bd2_ref, out_ref, slab, spad1, spad2,
                    *, H, W, C, lvl_shapes, bd_groups):
    zrow = jnp.zeros((1, W + 2, C), jnp.bfloat16)
    zcol = jnp.zeros((H + 2, 1, C), jnp.bfloat16)

    def project(x_ref, p_ref):
        return jnp.dot(x_ref[0], p_ref[...],
                       preferred_element_type=jnp.float32)

    slab[...] = (project(x0_ref, p0_ref) + bsum_ref[...]).reshape(H, W, C)

    for x_ref, p_ref, bd_ref, G, (hs, ws) in (
            (x1_ref, p1_ref, bd1_ref, bd_groups[0], lvl_shapes[0]),
            (x2_ref, p2_ref, bd2_ref, bd_groups[1], lvl_shapes[1])):
        fh = H // hs
        xp = project(x_ref, p_ref)
        y = _up_rows(xp.reshape(hs, ws, C), fh)
        _add_up_cols(slab, y, bd_ref, G, W, H, ws, C)

    spad1[0:1, :, :] = zrow
    spad1[H + 1:H + 2, :, :] = zrow
    spad1[:, 0:1, :] = zcol
    spad1[:, W + 1:W + 2, :] = zcol
    spad1[1:H + 1, 1:W + 1, :] = slab[...].astype(jnp.bfloat16)

    spad2[0:1, :, :] = zrow
    spad2[H + 1:H + 2, :, :] = zrow
    spad2[:, 0:1, :] = zcol
    spad2[:, W + 1:W + 2, :] = zcol

    th = 16
    nt = H // th

    def conv1_body(t, _):
        r0 = pl.multiple_of(t * th, th)
        y1 = jnp.maximum(_conv3x3_tile(spad1, w1_ref, r0, th, W, C)
                         + b1_ref[...], 0.0)
        spad2[pl.ds(r0 + 1, th), 1:W + 1, :] = \
            y1.reshape(th, W, C).astype(jnp.bfloat16)
        return 0

    jax.lax.fori_loop(0, nt, conv1_body, 0, unroll=False)

    def conv2_body(t, _):
        r0 = pl.multiple_of(t * th, th)
        y2 = _conv3x3_tile(spad2, w2_ref, r0, th, W, C) + b2_ref[...]
        out_ref[0, :, pl.ds(pl.multiple_of(r0 * W, th * W), th * W)] = \
            jnp.transpose(y2, (1, 0))
        return 0

    jax.lax.fori_loop(0, nt, conv2_body, 0, unroll=False)


def kernel(feat0, feat1, feat2, proj0_w, proj0_b, proj1_w, proj1_b, proj2_w,
           proj2_b, refine1_w, refine1_b, refine2_w, refine2_b):
    n, c0, H, W = feat0.shape
    c1, (h1, w1) = feat1.shape[1], feat1.shape[2:]
    c2, (h2, w2) = feat2.shape[1], feat2.shape[2:]
    C = refine1_b.shape[0]
    bf = jnp.bfloat16

    x0 = jnp.transpose(feat0, (0, 2, 3, 1)).reshape(n, H * W, c0).astype(bf)
    x1 = jnp.transpose(feat1, (0, 2, 3, 1)).reshape(n, h1 * w1, c1).astype(bf)
    x2 = jnp.transpose(feat2, (0, 2, 3, 1)).reshape(n, h2 * w2, c2).astype(bf)

    bsum = (proj0_b + proj1_b + proj2_b).astype(jnp.float32).reshape(1, C)
    wk1 = refine1_w.reshape(9 * C, C).astype(bf)
    wk2 = refine2_w.reshape(9 * C, C).astype(bf)

    def bd_interp(ws):
        g = max(1, min(H, 512 // ws))
        return g, jnp.asarray(np.kron(np.eye(g, dtype=np.float32),
                                      _interp_matrix(W, ws))).astype(bf)

    g1, bd1 = bd_interp(w1)
    g2, bd2 = bd_interp(w2)

    inputs = [x0, x1, x2, proj0_w.astype(bf), proj1_w.astype(bf),
              proj2_w.astype(bf), bsum, wk1,
              refine1_b.astype(jnp.float32).reshape(1, C), wk2,
              refine2_b.astype(jnp.float32).reshape(1, C), bd1, bd2]

    def bspec(shape, bmap):
        return pl.BlockSpec(shape, bmap)

    batch0 = lambda b: (b, 0, 0)
    const2 = lambda b: (0, 0)
    in_specs = [
        bspec((1, H * W, c0), batch0),
        bspec((1, h1 * w1, c1), batch0),
        bspec((1, h2 * w2, c2), batch0),
        bspec((c0, C), const2), bspec((c1, C), const2), bspec((c2, C), const2),
        bspec((1, C), const2),
        bspec((9 * C, C), const2), bspec((1, C), const2),
        bspec((9 * C, C), const2), bspec((1, C), const2),
        bspec((g1 * W, g1 * w1), const2), bspec((g2 * W, g2 * w2), const2),
    ]

    kfn = functools.partial(_decoder_kernel, H=H, W=W, C=C,
                            lvl_shapes=((h1, w1), (h2, w2)),
                            bd_groups=(g1, g2))
    flops = 2 * n * (H * W * c0 * C + h1 * w1 * c1 * C + h2 * w2 * c2 * C
                     + 2 * 9 * H * W * C * C)
    in_bytes = sum(int(np.prod(a.shape)) * a.dtype.itemsize for a in inputs)
    out_bytes = 4 * n * C * H * W

    out = pl.pallas_call(
        kfn,
        out_shape=jax.ShapeDtypeStruct((n, C, H * W), jnp.float32),
        grid=(n,),
        in_specs=in_specs,
        out_specs=pl.BlockSpec((1, C, H * W), lambda b: (b, 0, 0)),
        scratch_shapes=[pltpu.VMEM((H, W, C), jnp.float32),
                        pltpu.VMEM((H + 2, W + 2, C), bf),
                        pltpu.VMEM((H + 2, W + 2, C), bf)],
        compiler_params=pltpu.CompilerParams(
            dimension_semantics=("parallel",),
            vmem_limit_bytes=60 * 1024 * 1024),
        cost_estimate=pl.CostEstimate(flops=int(flops), transcendentals=0,
                                      bytes_accessed=int(in_bytes + out_bytes)),
    )(*inputs)
    return out.reshape(n, C, H, W)


# speedup vs baseline: 1.0370x; 1.0370x over previous
import functools

import numpy as np
import jax
import jax.numpy as jnp
from jax.experimental import pallas as pl
from jax.experimental.pallas import tpu as pltpu


def _phases(f):
    out = []
    for k in range(f):
        d = (k + 0.5) / f - 0.5
        if d < 0:
            out.append((True, -d, 1.0 + d))
        else:
            out.append((False, 1.0 - d, d))
    return out


def _up_rows(x, f):
    if f == 1:
        return x
    xprev = jnp.concatenate([x[:1], x[:-1]], axis=0)
    xnext = jnp.concatenate([x[1:], x[-1:]], axis=0)
    cols = []
    for use_prev, wa, wb in _phases(f):
        cols.append(wa * xprev + wb * x if use_prev else wa * x + wb * xnext)
    y = jnp.stack(cols, axis=1)
    return y.reshape(x.shape[0] * f, x.shape[1], x.shape[2])


def _add_up_cols(slab, y, f, ws):
    if f == 1:
        slab[...] = slab[...] + y
        return
    yprev = jnp.concatenate([y[:, :1], y[:, :-1]], axis=1)
    ynext = jnp.concatenate([y[:, 1:], y[:, -1:]], axis=1)
    for k, (use_prev, wa, wb) in enumerate(_phases(f)):
        ph = wa * yprev + wb * y if use_prev else wa * y + wb * ynext
        idx = (slice(None), pl.Slice(k, ws, f), slice(None))
        slab[idx] = slab[idx] + ph


def _conv3x3_tile(spad, w_ref, r0, th, W, C):
    acc = jnp.zeros((th * W, C), jnp.float32)
    for dh in range(3):
        for dw in range(3):
            k = dh * 3 + dw
            xs = spad[pl.ds(r0 + dh, th), dw:dw + W, :].reshape(th * W, C)
            acc = acc + jnp.dot(xs, w_ref[k * C:(k + 1) * C, :],
                                preferred_element_type=jnp.float32)
    return acc


def _decoder_kernel(x0_ref, x1_ref, x2_ref, p0_ref, p1_ref, p2_ref,
                    bsum_ref, w1_ref, b1_ref, w2_ref, b2_ref,
                    out_ref, slab, spad1, spad2, *, H, W, C, lvl_shapes):
    zrow = jnp.zeros((1, W + 2, C), jnp.bfloat16)
    zcol = jnp.zeros((H + 2, 1, C), jnp.bfloat16)

    def project(x_ref, p_ref):
        return jnp.dot(x_ref[0], p_ref[...],
                       preferred_element_type=jnp.float32)

    slab[...] = (project(x0_ref, p0_ref) + bsum_ref[...]).reshape(H, W, C)

    for x_ref, p_ref, (hs, ws) in ((x1_ref, p1_ref, lvl_shapes[0]),
                                   (x2_ref, p2_ref, lvl_shapes[1])):
        fh, fw = H // hs, W // ws
        xp = project(x_ref, p_ref)
        y = _up_rows(xp.reshape(hs, ws, C), fh)
        _add_up_cols(slab, y, fw, ws)

    spad1[0:1, :, :] = zrow
    spad1[H + 1:H + 2, :, :] = zrow
    spad1[:, 0:1, :] = zcol
    spad1[:, W + 1:W + 2, :] = zcol
    spad1[1:H + 1, 1:W + 1, :] = slab[...].astype(jnp.bfloat16)

    spad2[0:1, :, :] = zrow
    spad2[H + 1:H + 2, :, :] = zrow
    spad2[:, 0:1, :] = zcol
    spad2[:, W + 1:W + 2, :] = zcol

    th = 8
    nt = H // th

    def conv1_body(t, _):
        r0 = pl.multiple_of(t * th, th)
        y1 = jnp.maximum(_conv3x3_tile(spad1, w1_ref, r0, th, W, C)
                         + b1_ref[...], 0.0)
        spad2[pl.ds(r0 + 1, th), 1:W + 1, :] = \
            y1.reshape(th, W, C).astype(jnp.bfloat16)
        return 0

    jax.lax.fori_loop(0, nt, conv1_body, 0, unroll=False)

    def conv2_body(t, _):
        r0 = pl.multiple_of(t * th, th)
        y2 = _conv3x3_tile(spad2, w2_ref, r0, th, W, C) + b2_ref[...]
        out_ref[0, :, pl.ds(pl.multiple_of(r0 * W, th * W), th * W)] = \
            jnp.transpose(y2, (1, 0))
        return 0

    jax.lax.fori_loop(0, nt, conv2_body, 0, unroll=False)


def kernel(feat0, feat1, feat2, proj0_w, proj0_b, proj1_w, proj1_b, proj2_w,
           proj2_b, refine1_w, refine1_b, refine2_w, refine2_b):
    n, c0, H, W = feat0.shape
    c1, (h1, w1) = feat1.shape[1], feat1.shape[2:]
    c2, (h2, w2) = feat2.shape[1], feat2.shape[2:]
    C = refine1_b.shape[0]
    bf = jnp.bfloat16

    x0 = jnp.transpose(feat0, (0, 2, 3, 1)).reshape(n, H * W, c0).astype(bf)
    x1 = jnp.transpose(feat1, (0, 2, 3, 1)).reshape(n, h1 * w1, c1).astype(bf)
    x2 = jnp.transpose(feat2, (0, 2, 3, 1)).reshape(n, h2 * w2, c2).astype(bf)

    bsum = (proj0_b + proj1_b + proj2_b).astype(jnp.float32).reshape(1, C)
    wk1 = refine1_w.reshape(9 * C, C).astype(bf)
    wk2 = refine2_w.reshape(9 * C, C).astype(bf)

    inputs = [x0, x1, x2, proj0_w.astype(bf), proj1_w.astype(bf),
              proj2_w.astype(bf), bsum, wk1,
              refine1_b.astype(jnp.float32).reshape(1, C), wk2,
              refine2_b.astype(jnp.float32).reshape(1, C)]

    def bspec(shape, bmap):
        return pl.BlockSpec(shape, bmap)

    batch0 = lambda b: (b, 0, 0)
    const2 = lambda b: (0, 0)
    in_specs = [
        bspec((1, H * W, c0), batch0),
        bspec((1, h1 * w1, c1), batch0),
        bspec((1, h2 * w2, c2), batch0),
        bspec((c0, C), const2), bspec((c1, C), const2), bspec((c2, C), const2),
        bspec((1, C), const2),
        bspec((9 * C, C), const2), bspec((1, C), const2),
        bspec((9 * C, C), const2), bspec((1, C), const2),
    ]

    kfn = functools.partial(_decoder_kernel, H=H, W=W, C=C,
                            lvl_shapes=((h1, w1), (h2, w2)))
    flops = 2 * n * (H * W * c0 * C + h1 * w1 * c1 * C + h2 * w2 * c2 * C
                     + 2 * 9 * H * W * C * C)
    in_bytes = sum(int(np.prod(a.shape)) * a.dtype.itemsize for a in inputs)
    out_bytes = 4 * n * C * H * W

    out = pl.pallas_call(
        kfn,
        out_shape=jax.ShapeDtypeStruct((n, C, H * W), jnp.float32),
        grid=(n,),
        in_specs=in_specs,
        out_specs=pl.BlockSpec((1, C, H * W), lambda b: (b, 0, 0)),
        scratch_shapes=[pltpu.VMEM((H, W, C), jnp.float32),
                        pltpu.VMEM((H + 2, W + 2, C), bf),
                        pltpu.VMEM((H + 2, W + 2, C), bf)],
        compiler_params=pltpu.CompilerParams(
            dimension_semantics=("parallel",),
            vmem_limit_bytes=60 * 1024 * 1024),
        cost_estimate=pl.CostEstimate(flops=int(flops), transcendentals=0,
                                      bytes_accessed=int(in_bytes + out_bytes)),
    )(*inputs)
    return out.reshape(n, C, H, W)


# speedup vs baseline: 1.0701x; 1.0319x over previous
import functools

import numpy as np
import jax
import jax.numpy as jnp
from jax.experimental import pallas as pl
from jax.experimental.pallas import tpu as pltpu


def _phases(f):
    out = []
    for k in range(f):
        d = (k + 0.5) / f - 0.5
        if d < 0:
            out.append((True, -d, 1.0 + d))
        else:
            out.append((False, 1.0 - d, d))
    return out


def _up_rows(x, f):
    if f == 1:
        return x
    xprev = jnp.concatenate([x[:1], x[:-1]], axis=0)
    xnext = jnp.concatenate([x[1:], x[-1:]], axis=0)
    cols = []
    for use_prev, wa, wb in _phases(f):
        cols.append(wa * xprev + wb * x if use_prev else wa * x + wb * xnext)
    y = jnp.stack(cols, axis=1)
    return y.reshape(x.shape[0] * f, x.shape[1], x.shape[2])


def _add_up_cols(slab, y, f, ws):
    if f == 1:
        slab[...] = slab[...] + y
        return
    yprev = jnp.concatenate([y[:, :1], y[:, :-1]], axis=1)
    ynext = jnp.concatenate([y[:, 1:], y[:, -1:]], axis=1)
    for k, (use_prev, wa, wb) in enumerate(_phases(f)):
        ph = wa * yprev + wb * y if use_prev else wa * y + wb * ynext
        idx = (slice(None), pl.Slice(k, ws, f), slice(None))
        slab[idx] = slab[idx] + ph


def _conv3x3_tile(spad, w_ref, r0, th, W, C):
    acc = jnp.zeros((th * W, C), jnp.float32)
    for dh in range(3):
        for dw in range(3):
            k = dh * 3 + dw
            xs = spad[pl.ds(r0 + dh, th), dw:dw + W, :].reshape(th * W, C)
            acc = acc + jnp.dot(xs, w_ref[k * C:(k + 1) * C, :],
                                preferred_element_type=jnp.float32)
    return acc


def _decoder_kernel(x0_ref, x1_ref, x2_ref, p0_ref, p1_ref, p2_ref,
                    bsum_ref, w1_ref, b1_ref, w2_ref, b2_ref,
                    out_ref, slab, spad1, spad2, *, H, W, C, lvl_shapes):
    zrow = jnp.zeros((1, W + 2, C), jnp.bfloat16)
    zcol = jnp.zeros((H + 2, 1, C), jnp.bfloat16)

    def project(x_ref, p_ref):
        return jnp.dot(x_ref[0], p_ref[...],
                       preferred_element_type=jnp.float32)

    slab[...] = (project(x0_ref, p0_ref) + bsum_ref[...]).reshape(H, W, C)

    for x_ref, p_ref, (hs, ws) in ((x1_ref, p1_ref, lvl_shapes[0]),
                                   (x2_ref, p2_ref, lvl_shapes[1])):
        fh, fw = H // hs, W // ws
        xp = project(x_ref, p_ref)
        y = _up_rows(xp.reshape(hs, ws, C), fh)
        _add_up_cols(slab, y, fw, ws)

    spad1[0:1, :, :] = zrow
    spad1[H + 1:H + 2, :, :] = zrow
    spad1[:, 0:1, :] = zcol
    spad1[:, W + 1:W + 2, :] = zcol
    spad1[1:H + 1, 1:W + 1, :] = slab[...].astype(jnp.bfloat16)

    spad2[0:1, :, :] = zrow
    spad2[H + 1:H + 2, :, :] = zrow
    spad2[:, 0:1, :] = zcol
    spad2[:, W + 1:W + 2, :] = zcol

    th = 16
    nt = H // th

    def conv1_body(t, _):
        r0 = pl.multiple_of(t * th, th)
        y1 = jnp.maximum(_conv3x3_tile(spad1, w1_ref, r0, th, W, C)
                         + b1_ref[...], 0.0)
        spad2[pl.ds(r0 + 1, th), 1:W + 1, :] = \
            y1.reshape(th, W, C).astype(jnp.bfloat16)
        return 0

    jax.lax.fori_loop(0, nt, conv1_body, 0, unroll=False)

    def conv2_body(t, _):
        r0 = pl.multiple_of(t * th, th)
        y2 = _conv3x3_tile(spad2, w2_ref, r0, th, W, C) + b2_ref[...]
        out_ref[0, :, pl.ds(pl.multiple_of(r0 * W, th * W), th * W)] = \
            jnp.transpose(y2, (1, 0))
        return 0

    jax.lax.fori_loop(0, nt, conv2_body, 0, unroll=False)


def kernel(feat0, feat1, feat2, proj0_w, proj0_b, proj1_w, proj1_b, proj2_w,
           proj2_b, refine1_w, refine1_b, refine2_w, refine2_b):
    n, c0, H, W = feat0.shape
    c1, (h1, w1) = feat1.shape[1], feat1.shape[2:]
    c2, (h2, w2) = feat2.shape[1], feat2.shape[2:]
    C = refine1_b.shape[0]
    bf = jnp.bfloat16

    x0 = jnp.transpose(feat0, (0, 2, 3, 1)).reshape(n, H * W, c0).astype(bf)
    x1 = jnp.transpose(feat1, (0, 2, 3, 1)).reshape(n, h1 * w1, c1).astype(bf)
    x2 = jnp.transpose(feat2, (0, 2, 3, 1)).reshape(n, h2 * w2, c2).astype(bf)

    bsum = (proj0_b + proj1_b + proj2_b).astype(jnp.float32).reshape(1, C)
    wk1 = refine1_w.reshape(9 * C, C).astype(bf)
    wk2 = refine2_w.reshape(9 * C, C).astype(bf)

    inputs = [x0, x1, x2, proj0_w.astype(bf), proj1_w.astype(bf),
              proj2_w.astype(bf), bsum, wk1,
              refine1_b.astype(jnp.float32).reshape(1, C), wk2,
              refine2_b.astype(jnp.float32).reshape(1, C)]

    def bspec(shape, bmap):
        return pl.BlockSpec(shape, bmap)

    batch0 = lambda b: (b, 0, 0)
    const2 = lambda b: (0, 0)
    in_specs = [
        bspec((1, H * W, c0), batch0),
        bspec((1, h1 * w1, c1), batch0),
        bspec((1, h2 * w2, c2), batch0),
        bspec((c0, C), const2), bspec((c1, C), const2), bspec((c2, C), const2),
        bspec((1, C), const2),
        bspec((9 * C, C), const2), bspec((1, C), const2),
        bspec((9 * C, C), const2), bspec((1, C), const2),
    ]

    kfn = functools.partial(_decoder_kernel, H=H, W=W, C=C,
                            lvl_shapes=((h1, w1), (h2, w2)))
    flops = 2 * n * (H * W * c0 * C + h1 * w1 * c1 * C + h2 * w2 * c2 * C
                     + 2 * 9 * H * W * C * C)
    in_bytes = sum(int(np.prod(a.shape)) * a.dtype.itemsize for a in inputs)
    out_bytes = 4 * n * C * H * W

    out = pl.pallas_call(
        kfn,
        out_shape=jax.ShapeDtypeStruct((n, C, H * W), jnp.float32),
        grid=(n,),
        in_specs=in_specs,
        out_specs=pl.BlockSpec((1, C, H * W), lambda b: (b, 0, 0)),
        scratch_shapes=[pltpu.VMEM((H, W, C), jnp.float32),
                        pltpu.VMEM((H + 2, W + 2, C), bf),
                        pltpu.VMEM((H + 2, W + 2, C), bf)],
        compiler_params=pltpu.CompilerParams(
            dimension_semantics=("parallel",),
            vmem_limit_bytes=60 * 1024 * 1024),
        cost_estimate=pl.CostEstimate(flops=int(flops), transcendentals=0,
                                      bytes_accessed=int(in_bytes + out_bytes)),
    )(*inputs)
    return out.reshape(n, C, H, W)
